# baseline (speedup 1.0000x reference)
import jax
import jax.numpy as jnp
from jax.experimental import pallas as pl
from jax.experimental.pallas import tpu as pltpu

N = 100000
B = 64
D = 128
M = 2048
RADIUS = 1.0

BLK = 4096
CPB = 16


def _encode(pos_ref, batch_ref, w1_ref, w2_ref, b2_ref, enc_ref):
    i = pl.program_id(0)

    @pl.when(i == 0)
    def _init():
        enc_ref[...] = jnp.zeros_like(enc_ref)

    base = i * BLK
    valid = (jax.lax.broadcasted_iota(jnp.int32, (BLK, 1), 0) + base) < N
    x = jnp.where(valid, pos_ref[...], 0.0)
    x4 = jnp.concatenate(
        [x, jnp.ones((BLK, 1), jnp.float32)], axis=1)

    h = jnp.dot(x4.astype(jnp.bfloat16), w1_ref[...],
                preferred_element_type=jnp.float32)
    h = jnp.maximum(h.astype(jnp.bfloat16), jnp.bfloat16(0))
    h = jnp.dot(h, w2_ref[...],
                preferred_element_type=jnp.float32)
    h = jnp.maximum(h.astype(jnp.bfloat16) + b2_ref[...], jnp.bfloat16(0))

    seg = batch_ref[...]
    rows = jax.lax.broadcasted_iota(jnp.int32, (B, BLK), 0)
    onehot = (rows == seg).astype(jnp.bfloat16)
    enc_ref[...] += jnp.dot(onehot, h,
                            preferred_element_type=jnp.float32)


def _decode(enc_ref, v0_ref, v1_ref, v2_ref, db_ref, pts_ref,
            x0_ref, x1_ref, x2_ref):
    i = pl.program_id(0)

    @pl.when(i == 0)
    def _mm():
        enc = enc_ref[...]
        x0_ref[...] = (jnp.dot(enc, v0_ref[...],
                               preferred_element_type=jnp.float32)
                       + db_ref[0:1, :]) * RADIUS
        x1_ref[...] = (jnp.dot(enc, v1_ref[...],
                               preferred_element_type=jnp.float32)
                       + db_ref[1:2, :]) * RADIUS
        x2_ref[...] = (jnp.dot(enc, v2_ref[...],
                               preferred_element_type=jnp.float32)
                       + db_ref[2:3, :]) * RADIUS

    cols = []
    for j in range(CPB):
        b = i * CPB + j
        cols.append(jnp.concatenate(
            [x0_ref[pl.ds(b, 1), :].reshape(M, 1),
             x1_ref[pl.ds(b, 1), :].reshape(M, 1),
             x2_ref[pl.ds(b, 1), :].reshape(M, 1)], axis=1))
    pts_ref[...] = jnp.concatenate(cols, axis=0)


def kernel(pos, batch, enc_W1, enc_b1, enc_W2, enc_b2, dec_W, dec_b):
    n = pos.shape[0]
    grid = (n + BLK - 1) // BLK
    npad = grid * BLK

    w1a = jnp.concatenate([enc_W1 * (1.0 / RADIUS), enc_b1[None, :]],
                          axis=0).astype(jnp.bfloat16)
    batch_pad = jnp.concatenate(
        [batch, jnp.full((npad - n,), B, jnp.int32)]).reshape(1, npad)
    v = dec_W.reshape(D, M, 3)
    db3 = dec_b.reshape(M, 3).T

    enc = pl.pallas_call(
        _encode,
        grid=(grid,),
        in_specs=[
            pl.BlockSpec((BLK, 3), lambda i: (i, 0)),
            pl.BlockSpec((1, BLK), lambda i: (0, i)),
            pl.BlockSpec((4, D), lambda i: (0, 0)),
            pl.BlockSpec((D, D), lambda i: (0, 0)),
            pl.BlockSpec((1, D), lambda i: (0, 0)),
        ],
        out_specs=pl.BlockSpec((B, D), lambda i: (0, 0)),
        out_shape=jax.ShapeDtypeStruct((B, D), jnp.float32),
    )(pos, batch_pad, w1a, enc_W2.astype(jnp.bfloat16),
      enc_b2.reshape(1, D).astype(jnp.bfloat16))

    pts = pl.pallas_call(
        _decode,
        grid=(B // CPB,),
        in_specs=[
            pl.BlockSpec((B, D), lambda i: (0, 0)),
            pl.BlockSpec((D, M), lambda i: (0, 0)),
            pl.BlockSpec((D, M), lambda i: (0, 0)),
            pl.BlockSpec((D, M), lambda i: (0, 0)),
            pl.BlockSpec((3, M), lambda i: (0, 0)),
        ],
        out_specs=pl.BlockSpec((CPB * M, 3), lambda i: (i, 0)),
        out_shape=jax.ShapeDtypeStruct((B * M, 3), jnp.float32),
        scratch_shapes=[pltpu.VMEM((B, M), jnp.float32)] * 3,
    )(enc, v[:, :, 0], v[:, :, 1], v[:, :, 2], db3)

    bout = jnp.repeat(jnp.arange(B, dtype=jnp.int32), M)
    return (pos, batch, pts, bout)


# speedup vs baseline: 1.0150x; 1.0150x over previous
import jax
import jax.numpy as jnp
from jax.experimental import pallas as pl
from jax.experimental.pallas import tpu as pltpu

N = 100000
B = 64
D = 128
M = 2048
RADIUS = 1.0

BLK = 4096
CPB = 4
QD = 4


def _encode(pos_ref, batch_ref, w1_ref, w2_ref, b2_ref, enc_ref):
    i = pl.program_id(0)

    @pl.when(i == 0)
    def _init():
        enc_ref[...] = jnp.zeros_like(enc_ref)

    base = i * BLK
    valid = (jax.lax.broadcasted_iota(jnp.int32, (BLK, 1), 0) + base) < N
    x = jnp.where(valid, pos_ref[...], 0.0)
    x4 = jnp.concatenate(
        [x, jnp.ones((BLK, 1), jnp.float32)], axis=1)

    h = jnp.dot(x4.astype(jnp.bfloat16), w1_ref[...],
                preferred_element_type=jnp.float32)
    h = jnp.maximum(h.astype(jnp.bfloat16), jnp.bfloat16(0))
    h = jnp.dot(h, w2_ref[...],
                preferred_element_type=jnp.float32)
    h = jnp.maximum(h.astype(jnp.bfloat16) + b2_ref[...], jnp.bfloat16(0))

    seg = batch_ref[...]
    rows = jax.lax.broadcasted_iota(jnp.int32, (B, BLK), 0)
    onehot = (rows == seg).astype(jnp.bfloat16)
    enc_ref[...] += jnp.dot(onehot, h,
                            preferred_element_type=jnp.float32)


def _decode(enc_ref, v0_ref, v1_ref, v2_ref, db_ref, pts_ref,
            x0_ref, x1_ref, x2_ref, buf_ref, sems):
    i = pl.program_id(0)
    nsteps = pl.num_programs(0)

    @pl.when(i == 0)
    def _mm():
        enc = enc_ref[...]
        x0_ref[...] = (jnp.dot(enc, v0_ref[...],
                               preferred_element_type=jnp.float32)
                       + db_ref[0:1, :]) * RADIUS
        x1_ref[...] = (jnp.dot(enc, v1_ref[...],
                               preferred_element_type=jnp.float32)
                       + db_ref[1:2, :]) * RADIUS
        x2_ref[...] = (jnp.dot(enc, v2_ref[...],
                               preferred_element_type=jnp.float32)
                       + db_ref[2:3, :]) * RADIUS

    slot = jax.lax.rem(i, QD)

    @pl.when(i >= QD)
    def _reclaim():
        pltpu.make_async_copy(
            buf_ref.at[slot], pts_ref.at[pl.ds(i * CPB * M, CPB * M), :],
            sems.at[slot]).wait()

    cols = []
    for j in range(CPB):
        b = i * CPB + j
        cols.append(jnp.concatenate(
            [x0_ref[pl.ds(b, 1), :].reshape(M, 1),
             x1_ref[pl.ds(b, 1), :].reshape(M, 1),
             x2_ref[pl.ds(b, 1), :].reshape(M, 1)], axis=1))
    buf_ref[slot] = jnp.concatenate(cols, axis=0)
    pltpu.make_async_copy(
        buf_ref.at[slot], pts_ref.at[pl.ds(i * CPB * M, CPB * M), :],
        sems.at[slot]).start()

    @pl.when(i == nsteps - 1)
    def _drain():
        for s in range(QD):
            pltpu.make_async_copy(
                buf_ref.at[s], pts_ref.at[pl.ds(s * CPB * M, CPB * M), :],
                sems.at[s]).wait()


def kernel(pos, batch, enc_W1, enc_b1, enc_W2, enc_b2, dec_W, dec_b):
    n = pos.shape[0]
    grid = (n + BLK - 1) // BLK
    npad = grid * BLK

    w1a = jnp.concatenate([enc_W1 * (1.0 / RADIUS), enc_b1[None, :]],
                          axis=0).astype(jnp.bfloat16)
    batch_pad = jnp.concatenate(
        [batch, jnp.full((npad - n,), B, jnp.int32)]).reshape(1, npad)
    v = dec_W.reshape(D, M, 3)
    db3 = dec_b.reshape(M, 3).T

    enc = pl.pallas_call(
        _encode,
        grid=(grid,),
        in_specs=[
            pl.BlockSpec((BLK, 3), lambda i: (i, 0)),
            pl.BlockSpec((1, BLK), lambda i: (0, i)),
            pl.BlockSpec((4, D), lambda i: (0, 0)),
            pl.BlockSpec((D, D), lambda i: (0, 0)),
            pl.BlockSpec((1, D), lambda i: (0, 0)),
        ],
        out_specs=pl.BlockSpec((B, D), lambda i: (0, 0)),
        out_shape=jax.ShapeDtypeStruct((B, D), jnp.float32),
    )(pos, batch_pad, w1a, enc_W2.astype(jnp.bfloat16),
      enc_b2.reshape(1, D).astype(jnp.bfloat16))

    pts = pl.pallas_call(
        _decode,
        grid=(B // CPB,),
        in_specs=[
            pl.BlockSpec((B, D), lambda i: (0, 0)),
            pl.BlockSpec((D, M), lambda i: (0, 0)),
            pl.BlockSpec((D, M), lambda i: (0, 0)),
            pl.BlockSpec((D, M), lambda i: (0, 0)),
            pl.BlockSpec((3, M), lambda i: (0, 0)),
        ],
        out_specs=pl.BlockSpec(memory_space=pltpu.MemorySpace.HBM),
        out_shape=jax.ShapeDtypeStruct((B * M, 3), jnp.float32),
        scratch_shapes=[pltpu.VMEM((B, M), jnp.float32)] * 3
        + [pltpu.VMEM((QD, CPB * M, 3), jnp.float32),
           pltpu.SemaphoreType.DMA((QD,))],
    )(enc, v[:, :, 0], v[:, :, 1], v[:, :, 2], db3)

    bout = jnp.repeat(jnp.arange(B, dtype=jnp.int32), M)
    return (pos, batch, pts, bout)


# speedup vs baseline: 1.1147x; 1.0982x over previous
import jax
import jax.numpy as jnp
from jax.experimental import pallas as pl
from jax.experimental.pallas import tpu as pltpu

N = 100000
B = 64
D = 128
M = 2048
RADIUS = 1.0

BLK = 8192
CPB = 4


def _encode(pos_ref, batch_ref, w1_ref, w2_ref, b2_ref, enc_ref):
    i = pl.program_id(0)

    @pl.when(i == 0)
    def _init():
        enc_ref[...] = jnp.zeros_like(enc_ref)

    base = i * BLK
    valid = (jax.lax.broadcasted_iota(jnp.int32, (BLK, 1), 0) + base) < N
    x = jnp.where(valid, pos_ref[...], 0.0)
    x4 = jnp.concatenate(
        [x, jnp.ones((BLK, 1), jnp.float32)], axis=1)

    h = jnp.dot(x4.astype(jnp.bfloat16), w1_ref[...],
                preferred_element_type=jnp.float32)
    h = jnp.maximum(h.astype(jnp.bfloat16), jnp.bfloat16(0))
    h = jnp.dot(h, w2_ref[...],
                preferred_element_type=jnp.float32)
    h = jnp.maximum(h.astype(jnp.bfloat16) + b2_ref[...], jnp.bfloat16(0))

    seg = batch_ref[...]
    rows = jax.lax.broadcasted_iota(jnp.int32, (B, BLK), 0)
    onehot = (rows == seg).astype(jnp.bfloat16)
    enc_ref[...] += jnp.dot(onehot, h,
                            preferred_element_type=jnp.float32)


def _decode(enc_ref, v_ref, db_ref, pts_ref, x0_ref, x1_ref, x2_ref):
    i = pl.program_id(0)

    @pl.when(i == 0)
    def _mm():
        enc = enc_ref[...]
        x0_ref[...] = (jnp.dot(enc, v_ref[0],
                               preferred_element_type=jnp.float32)
                       + db_ref[0:1, :]) * RADIUS
        x1_ref[...] = (jnp.dot(enc, v_ref[1],
                               preferred_element_type=jnp.float32)
                       + db_ref[1:2, :]) * RADIUS
        x2_ref[...] = (jnp.dot(enc, v_ref[2],
                               preferred_element_type=jnp.float32)
                       + db_ref[2:3, :]) * RADIUS

    cols = []
    for j in range(CPB):
        b = i * CPB + j
        cols.append(jnp.concatenate(
            [x0_ref[pl.ds(b, 1), :].reshape(M, 1),
             x1_ref[pl.ds(b, 1), :].reshape(M, 1),
             x2_ref[pl.ds(b, 1), :].reshape(M, 1)], axis=1))
    pts_ref[...] = jnp.concatenate(cols, axis=0)


def kernel(pos, batch, enc_W1, enc_b1, enc_W2, enc_b2, dec_W, dec_b):
    n = pos.shape[0]
    grid = (n + BLK - 1) // BLK
    npad = grid * BLK

    w1a = jnp.concatenate([enc_W1 * (1.0 / RADIUS), enc_b1[None, :]],
                          axis=0).astype(jnp.bfloat16)
    batch_pad = jnp.concatenate(
        [batch, jnp.full((npad - n,), B, jnp.int32)]).reshape(1, npad)
    v = jnp.transpose(dec_W.reshape(D, M, 3), (2, 0, 1))
    db3 = dec_b.reshape(M, 3).T

    enc = pl.pallas_call(
        _encode,
        grid=(grid,),
        in_specs=[
            pl.BlockSpec((BLK, 3), lambda i: (i, 0)),
            pl.BlockSpec((1, BLK), lambda i: (0, i)),
            pl.BlockSpec((4, D), lambda i: (0, 0)),
            pl.BlockSpec((D, D), lambda i: (0, 0)),
            pl.BlockSpec((1, D), lambda i: (0, 0)),
        ],
        out_specs=pl.BlockSpec((B, D), lambda i: (0, 0)),
        out_shape=jax.ShapeDtypeStruct((B, D), jnp.float32),
    )(pos, batch_pad, w1a, enc_W2.astype(jnp.bfloat16),
      enc_b2.reshape(1, D).astype(jnp.bfloat16))

    pts = pl.pallas_call(
        _decode,
        grid=(B // CPB,),
        in_specs=[
            pl.BlockSpec((B, D), lambda i: (0, 0)),
            pl.BlockSpec((3, D, M), lambda i: (0, 0, 0)),
            pl.BlockSpec((3, M), lambda i: (0, 0)),
        ],
        out_specs=pl.BlockSpec((CPB * M, 3), lambda i: (i, 0)),
        out_shape=jax.ShapeDtypeStruct((B * M, 3), jnp.float32),
        scratch_shapes=[pltpu.VMEM((B, M), jnp.float32)] * 3,
    )(enc, v, db3)

    bout = jnp.repeat(jnp.arange(B, dtype=jnp.int32), M)
    return (pos, batch, pts, bout)


# speedup vs baseline: 1.1202x; 1.0049x over previous
import jax
import jax.numpy as jnp
from jax.experimental import pallas as pl
from jax.experimental.pallas import tpu as pltpu

N = 100000
B = 64
D = 128
M = 2048
RADIUS = 1.0

BLK = 16384
CPB = 4


def _encode(pos_ref, batch_ref, w1_ref, w2_ref, b2_ref, enc_ref):
    i = pl.program_id(0)

    @pl.when(i == 0)
    def _init():
        enc_ref[...] = jnp.zeros_like(enc_ref)

    base = i * BLK
    valid = (jax.lax.broadcasted_iota(jnp.int32, (BLK, 1), 0) + base) < N
    x = jnp.where(valid, pos_ref[...], 0.0)
    x4 = jnp.concatenate(
        [x, jnp.ones((BLK, 1), jnp.float32)], axis=1)

    h = jnp.dot(x4.astype(jnp.bfloat16), w1_ref[...],
                preferred_element_type=jnp.float32)
    h = jnp.maximum(h.astype(jnp.bfloat16), jnp.bfloat16(0))
    h = jnp.dot(h, w2_ref[...],
                preferred_element_type=jnp.float32)
    h = jnp.maximum(h.astype(jnp.bfloat16) + b2_ref[...], jnp.bfloat16(0))

    seg = batch_ref[...]
    rows = jax.lax.broadcasted_iota(jnp.int32, (B, BLK), 0)
    onehot = (rows == seg).astype(jnp.bfloat16)
    enc_ref[...] += jnp.dot(onehot, h,
                            preferred_element_type=jnp.float32)


def _decode(enc_ref, v_ref, db_ref, pts_ref, x0_ref, x1_ref, x2_ref):
    i = pl.program_id(0)

    @pl.when(i == 0)
    def _mm():
        enc = enc_ref[...]
        x0_ref[...] = (jnp.dot(enc, v_ref[0],
                               preferred_element_type=jnp.float32)
                       + db_ref[0:1, :]) * RADIUS
        x1_ref[...] = (jnp.dot(enc, v_ref[1],
                               preferred_element_type=jnp.float32)
                       + db_ref[1:2, :]) * RADIUS
        x2_ref[...] = (jnp.dot(enc, v_ref[2],
                               preferred_element_type=jnp.float32)
                       + db_ref[2:3, :]) * RADIUS

    cols = []
    for j in range(CPB):
        b = i * CPB + j
        cols.append(jnp.concatenate(
            [x0_ref[pl.ds(b, 1), :].reshape(M, 1),
             x1_ref[pl.ds(b, 1), :].reshape(M, 1),
             x2_ref[pl.ds(b, 1), :].reshape(M, 1)], axis=1))
    pts_ref[...] = jnp.concatenate(cols, axis=0)


def kernel(pos, batch, enc_W1, enc_b1, enc_W2, enc_b2, dec_W, dec_b):
    n = pos.shape[0]
    grid = (n + BLK - 1) // BLK
    npad = grid * BLK

    w1a = jnp.concatenate([enc_W1 * (1.0 / RADIUS), enc_b1[None, :]],
                          axis=0).astype(jnp.bfloat16)
    batch_pad = jnp.concatenate(
        [batch, jnp.full((npad - n,), B, jnp.int32)]).reshape(1, npad)
    v = jnp.transpose(dec_W.reshape(D, M, 3), (2, 0, 1))
    db3 = dec_b.reshape(M, 3).T

    enc = pl.pallas_call(
        _encode,
        grid=(grid,),
        in_specs=[
            pl.BlockSpec((BLK, 3), lambda i: (i, 0)),
            pl.BlockSpec((1, BLK), lambda i: (0, i)),
            pl.BlockSpec((4, D), lambda i: (0, 0)),
            pl.BlockSpec((D, D), lambda i: (0, 0)),
            pl.BlockSpec((1, D), lambda i: (0, 0)),
        ],
        out_specs=pl.BlockSpec((B, D), lambda i: (0, 0)),
        out_shape=jax.ShapeDtypeStruct((B, D), jnp.float32),
    )(pos, batch_pad, w1a, enc_W2.astype(jnp.bfloat16),
      enc_b2.reshape(1, D).astype(jnp.bfloat16))

    pts = pl.pallas_call(
        _decode,
        grid=(B // CPB,),
        in_specs=[
            pl.BlockSpec((B, D), lambda i: (0, 0)),
            pl.BlockSpec((3, D, M), lambda i: (0, 0, 0)),
            pl.BlockSpec((3, M), lambda i: (0, 0)),
        ],
        out_specs=pl.BlockSpec((CPB * M, 3), lambda i: (i, 0)),
        out_shape=jax.ShapeDtypeStruct((B * M, 3), jnp.float32),
        scratch_shapes=[pltpu.VMEM((B, M), jnp.float32)] * 3,
    )(enc, v, db3)

    bout = jnp.repeat(jnp.arange(B, dtype=jnp.int32), M)
    return (pos, batch, pts, bout)
